# baseline (speedup 1.0000x reference)
import functools
import math

import jax
import jax.numpy as jnp
from jax import lax
from jax.experimental import pallas as pl
from jax.experimental.pallas import tpu as pltpu
from jax.experimental.pallas import tpu_sc as plsc

N_NODES = 50000
PAD = 50176
NT = 16
CPT = PAD // NT
NVEC = CPT // 16
NR = 32
RW = PAD // NR
E_REAL = 800000
M_EDGES = E_REAL + N_NODES
MPAD = 851968
EPT = MPAD // NT
SUB = 2048
HALF = MPAD // 2
HPT = HALF // NT
WSUB = 4096
WNSUB = EPT // WSUB
NSUB = EPT // SUB
F1 = 64


def _take16(v, idx):
    return v.at[idx].get(mode="promise_in_bounds")


def _mesh():
    return plsc.VectorSubcoreMesh(core_axis_name="c", subcore_axis_name="s")


def _scparams():
    return pltpu.CompilerParams(needs_layout_passes=False)


def _sort_body(k_keep, keys_hbm, idx_hbm, mask_hbm,
               spA_k, spA_v, spB_k, spB_v, sp_hist, sp_mask,
               kbuf, vbuf, posbuf, mibuf, onesbuf, zbuf, hbuf, hist, base, tmp16):
    c = lax.axis_index("c")
    t = lax.axis_index("s")

    @pl.when(c == 0)
    def _core0():
        iota = lax.iota(jnp.int32, 16)

        def fill_const(buf, val16):
            def step(i, _):
                buf[pl.ds(i * 16, 16)] = val16
                return 0
            lax.fori_loop(0, NVEC, step, 0)

        fill_const(onesbuf, jnp.full((16,), 1.0, jnp.float32))
        fill_const(zbuf, jnp.zeros((16,), jnp.float32))
        pltpu.sync_copy(zbuf, sp_mask.at[pl.ds(t * CPT, CPT)])
        pltpu.sync_copy(keys_hbm.at[pl.ds(t * CPT, CPT)], kbuf)

        def fill_v(i, _):
            vbuf[pl.ds(i * 16, 16)] = t * CPT + i * 16 + iota
            return 0
        lax.fori_loop(0, NVEC, fill_v, 0)
        plsc.subcore_barrier()

        def vec_ranks(kk, sh):
            d16 = ((kk >> sh) & 0xFF).astype(jnp.int32)
            skey = (d16 << 4) | iota
            sk, _ = plsc.sort_key_val(skey, iota)
            ds = sk >> 4
            lanes = sk & 15
            prev = jnp.where(iota == 0, 256, _take16(ds, jnp.maximum(iota - 1, 0)))
            nxt = jnp.where(iota == 15, 256, _take16(ds, jnp.minimum(iota + 1, 15)))
            isfirst = ds != prev
            islast = ds != nxt
            starts = plsc.cummax(jnp.where(isfirst, iota, 0))
            rank = iota - starts
            return ds, lanes, rank, islast

        for p in range(4):
            sh = 8 * p
            src_k, src_v = (spA_k, spA_v) if p % 2 == 0 else (spB_k, spB_v)
            dst_k, dst_v = (spB_k, spB_v) if p % 2 == 0 else (spA_k, spA_v)
            if p > 0:
                pltpu.sync_copy(src_k.at[pl.ds(t * CPT, CPT)], kbuf)
                pltpu.sync_copy(src_v.at[pl.ds(t * CPT, CPT)], vbuf)

            def zh(i, _):
                hist[pl.ds(i * 16, 16)] = jnp.zeros((16,), jnp.int32)
                return 0
            lax.fori_loop(0, 16, zh, 0)

            def hwalk(i, _):
                kk = kbuf[pl.ds(i * 16, 16)]
                ds, _lanes, rank, islast = vec_ranks(kk, sh)
                old = plsc.load_gather(hist, [ds])
                plsc.store_scatter(hist, [ds], old + rank + 1, mask=islast)
                return 0
            lax.fori_loop(0, NVEC, hwalk, 0)
            pltpu.sync_copy(hist, sp_hist.at[t])
            plsc.subcore_barrier()
            pltpu.sync_copy(sp_hist, hbuf)

            def scanstep(dv, carry):
                def acc_t(tt, a):
                    return a + hbuf[tt, pl.ds(dv * 16, 16)]
                totv = lax.fori_loop(0, 16, acc_t, jnp.zeros((16,), jnp.int32))
                cs = plsc.cumsum(totv)
                excl = cs - totv + carry
                pfxv = lax.fori_loop(0, t, acc_t, jnp.zeros((16,), jnp.int32))
                base[pl.ds(dv * 16, 16)] = excl + pfxv
                return carry + cs[15]
            lax.fori_loop(0, 16, scanstep, jnp.int32(0))

            def swalk(i, _):
                kk = kbuf[pl.ds(i * 16, 16)]
                ds, lanes, rank, islast = vec_ranks(kk, sh)
                bs = plsc.load_gather(base, [ds])
                pos_s = bs + rank
                plsc.store_scatter(base, [ds], pos_s + 1, mask=islast)
                plsc.store_scatter(tmp16, [lanes], pos_s)
                posbuf[pl.ds(i * 16, 16)] = tmp16[pl.ds(0, 16)]
                return 0
            lax.fori_loop(0, NVEC, swalk, 0)
            pltpu.sync_copy(kbuf, dst_k.at[posbuf])
            pltpu.sync_copy(vbuf, dst_v.at[posbuf])
            plsc.subcore_barrier()

        pltpu.sync_copy(spA_v.at[pl.ds(t * CPT, CPT)], vbuf)
        pltpu.sync_copy(vbuf, idx_hbm.at[pl.ds(t * CPT, CPT)])

        def mwalk(i, _):
            v16 = vbuf[pl.ds(i * 16, 16)]
            gpos = t * CPT + i * 16 + iota
            mibuf[pl.ds(i * 16, 16)] = jnp.where(gpos < k_keep, v16, N_NODES)
            return 0
        lax.fori_loop(0, NVEC, mwalk, 0)
        pltpu.sync_copy(onesbuf, sp_mask.at[mibuf])
        plsc.subcore_barrier()
        pltpu.sync_copy(sp_mask.at[pl.ds(t * CPT, CPT)], zbuf)
        pltpu.sync_copy(zbuf, mask_hbm.at[pl.ds(t * CPT, CPT)])


@functools.lru_cache(maxsize=None)
def _make_sort(k_keep):
    return pl.kernel(
        functools.partial(_sort_body, k_keep),
        out_type=(jax.ShapeDtypeStruct((PAD,), jnp.int32),
                  jax.ShapeDtypeStruct((PAD,), jnp.float32)),
        mesh=_mesh(),
        compiler_params=_scparams(),
        scratch_types=[
            pltpu.VMEM_SHARED((PAD,), jnp.uint32),
            pltpu.VMEM_SHARED((PAD,), jnp.int32),
            pltpu.VMEM_SHARED((PAD,), jnp.uint32),
            pltpu.VMEM_SHARED((PAD,), jnp.int32),
            pltpu.VMEM_SHARED((NT, 256), jnp.int32),
            pltpu.VMEM_SHARED((PAD,), jnp.float32),
            pltpu.VMEM((CPT,), jnp.uint32),
            pltpu.VMEM((CPT,), jnp.int32),
            pltpu.VMEM((CPT,), jnp.int32),
            pltpu.VMEM((CPT,), jnp.int32),
            pltpu.VMEM((CPT,), jnp.float32),
            pltpu.VMEM((CPT,), jnp.float32),
            pltpu.VMEM((NT, 256), jnp.int32),
            pltpu.VMEM((256,), jnp.int32),
            pltpu.VMEM((256,), jnp.int32),
            pltpu.VMEM((16,), jnp.int32),
        ],
        name=f"sc_topk_sort_{k_keep}",
    )


def _desc_keys(score):
    b = lax.bitcast_convert_type(score, jnp.uint32)
    asc = jnp.where(b >> 31 != 0, ~b, b | jnp.uint32(0x80000000))
    desc = ~asc
    return jnp.concatenate([desc, jnp.full((PAD - N_NODES,), 0xFFFFFFFF, jnp.uint32)])


def _sc_topk(score, k_keep):
    idx_pad, mask_pad = _make_sort(k_keep)(_desc_keys(score))
    return idx_pad[:k_keep], mask_pad


def _deg_body2(dst_hbm, deg_hbm, sp_deg, dbuf, vbuf, zbuf):
    c = lax.axis_index("c")
    t = lax.axis_index("s")

    @pl.when(c == 0)
    def _core0():
        def fill1(i, _):
            vbuf[pl.ds(i * 16, 16)] = jnp.full((16,), 1.0, jnp.float32)
            return 0
        lax.fori_loop(0, SUB // 16, fill1, 0)

        def fill0(i, _):
            zbuf[pl.ds(i * 16, 16)] = jnp.zeros((16,), jnp.float32)
            return 0
        lax.fori_loop(0, CPT // 16, fill0, 0)
        pltpu.sync_copy(zbuf, sp_deg.at[pl.ds(t * CPT, CPT)])
        plsc.subcore_barrier()

        def step(i, _):
            pltpu.sync_copy(dst_hbm.at[pl.ds(t * EPT + i * SUB, SUB)], dbuf)
            pltpu.sync_copy(vbuf, sp_deg.at[dbuf], add=True)
            return 0
        lax.fori_loop(0, NSUB, step, 0)
        plsc.subcore_barrier()
        pltpu.sync_copy(sp_deg.at[pl.ds(t * CPT, CPT)], zbuf)
        pltpu.sync_copy(zbuf, deg_hbm.at[pl.ds(t * CPT, CPT)])


@functools.lru_cache(maxsize=None)
def _make_deg():
    return pl.kernel(
        _deg_body2,
        out_type=jax.ShapeDtypeStruct((PAD,), jnp.float32),
        mesh=_mesh(),
        compiler_params=_scparams(),
        scratch_types=[
            pltpu.VMEM_SHARED((PAD,), jnp.float32),
            pltpu.VMEM((SUB,), jnp.int32),
            pltpu.VMEM((SUB,), jnp.float32),
            pltpu.VMEM((CPT,), jnp.float32),
        ],
        name="sc_deg_hist",
    )


def _bucket_of(d16, btab, iota):
    q = d16 >> 6
    tb = plsc.load_gather(btab, [q])
    return tb + jnp.where(d16 >= (tb + 1) * RW, 1, 0)


def _part_body2(dst_hbm, src_hbm, x_hbm, dis_hbm, btab_hbm,
                pval_hbm, pn_hbm, pdst_hbm, rstart_hbm,
                sp_hist, sp_val, sp_n, sp_dst, sp_x, sp_dis,
                dbuf, sbuf, posbuf, valbuf, nbuf, dsdbuf,
                btab, hist, base, tmp16, hbuf, rsbuf):
    c = lax.axis_index("c")
    t = lax.axis_index("s")

    @pl.when(c == 0)
    def _core0():
        iota = lax.iota(jnp.int32, 16)
        pltpu.sync_copy(btab_hbm, btab)

        @pl.when(t == 0)
        def _load_tabs():
            pltpu.sync_copy(x_hbm, sp_x)
            pltpu.sync_copy(dis_hbm, sp_dis)

        def zh(i, _):
            hist[pl.ds(i * 16, 16)] = jnp.zeros((16,), jnp.int32)
            return 0
        lax.fori_loop(0, 8, zh, 0)

        def vec_ranks32(b16):
            skey = (b16 << 4) | iota
            sk, _ = plsc.sort_key_val(skey, iota)
            ds = sk >> 4
            lanes = sk & 15
            prev = jnp.where(iota == 0, 64, _take16(ds, jnp.maximum(iota - 1, 0)))
            nxt = jnp.where(iota == 15, 64, _take16(ds, jnp.minimum(iota + 1, 15)))
            isfirst = ds != prev
            islast = ds != nxt
            starts = plsc.cummax(jnp.where(isfirst, iota, 0))
            rank = iota - starts
            return ds, lanes, rank, islast

        def hsub(i, _):
            pltpu.sync_copy(dst_hbm.at[pl.ds(t * EPT + i * WSUB, WSUB)], dbuf)

            def hwalk(j, _2):
                d16 = dbuf[pl.ds(j * 16, 16)]
                b16 = _bucket_of(d16, btab, iota)
                ds, _l, rank, islast = vec_ranks32(b16)
                old = plsc.load_gather(hist, [ds])
                plsc.store_scatter(hist, [ds], old + rank + 1, mask=islast)
                return 0
            lax.fori_loop(0, WSUB // 16, hwalk, 0)
            return 0
        lax.fori_loop(0, WNSUB, hsub, 0)
        pltpu.sync_copy(hist, sp_hist.at[t])
        plsc.subcore_barrier()
        pltpu.sync_copy(sp_hist, hbuf)

        def scanstep(bv, carry):
            def acc_t(tt, a):
                return a + hbuf[tt, pl.ds(bv * 16, 16)]
            totv = lax.fori_loop(0, NT, acc_t, jnp.zeros((16,), jnp.int32))
            cs = plsc.cumsum(totv)
            excl = cs - totv + carry
            pfxv = lax.fori_loop(0, t, acc_t, jnp.zeros((16,), jnp.int32))
            base[pl.ds(bv * 16, 16)] = excl + pfxv
            rsbuf[pl.ds(bv * 16, 16)] = excl
            return carry + cs[15]
        total = lax.fori_loop(0, 2, scanstep, jnp.int32(0))

        @pl.when(t == 0)
        def _wr_rstart():
            rsbuf[pl.ds(32, 16)] = jnp.full((16,), total, jnp.int32)
            pltpu.sync_copy(rsbuf, rstart_hbm)

        for wpass in range(2):
            w0 = wpass * HALF

            def rebase(bv, carry):
                def acc_t(tt, a):
                    return a + hbuf[tt, pl.ds(bv * 16, 16)]
                totv = lax.fori_loop(0, NT, acc_t, jnp.zeros((16,), jnp.int32))
                cs = plsc.cumsum(totv)
                excl = cs - totv + carry
                pfxv = lax.fori_loop(0, t, acc_t, jnp.zeros((16,), jnp.int32))
                base[pl.ds(bv * 16, 16)] = excl + pfxv
                return carry + cs[15]
            lax.fori_loop(0, 2, rebase, jnp.int32(0))

            def psub(i, _):
                pltpu.sync_copy(dst_hbm.at[pl.ds(t * EPT + i * WSUB, WSUB)], dbuf)
                pltpu.sync_copy(src_hbm.at[pl.ds(t * EPT + i * WSUB, WSUB)], sbuf)
                pltpu.sync_copy(sp_x.at[sbuf], valbuf)
                pltpu.sync_copy(sp_dis.at[sbuf], nbuf)
                pltpu.sync_copy(sp_dis.at[dbuf], dsdbuf)

                def pwalk(j, _2):
                    d16 = dbuf[pl.ds(j * 16, 16)]
                    b16 = _bucket_of(d16, btab, iota)
                    ds, lanes, rank, islast = vec_ranks32(b16)
                    bs = plsc.load_gather(base, [ds])
                    pos_s = bs + rank
                    plsc.store_scatter(base, [ds], pos_s + 1, mask=islast)
                    plsc.store_scatter(tmp16, [lanes], pos_s)
                    pos16 = tmp16[pl.ds(0, 16)]
                    loc = pos16 - w0
                    inwin = jnp.logical_and(pos16 >= w0, loc < HALF)
                    posbuf[pl.ds(j * 16, 16)] = jnp.where(inwin, loc, HALF)
                    nbuf[pl.ds(j * 16, 16)] = nbuf[pl.ds(j * 16, 16)] * dsdbuf[pl.ds(j * 16, 16)]
                    return 0
                lax.fori_loop(0, WSUB // 16, pwalk, 0)
                pltpu.sync_copy(valbuf, sp_val.at[posbuf])
                pltpu.sync_copy(nbuf, sp_n.at[posbuf])
                pltpu.sync_copy(dbuf, sp_dst.at[posbuf])
                return 0
            lax.fori_loop(0, WNSUB, psub, 0)
            plsc.subcore_barrier()

            def dump(i, _):
                off = t * HPT + i * SUB
                sv = valbuf.at[pl.ds(0, SUB)]
                sn = nbuf.at[pl.ds(0, SUB)]
                sd = dbuf.at[pl.ds(0, SUB)]
                pltpu.sync_copy(sp_val.at[pl.ds(off, SUB)], sv)
                pltpu.sync_copy(sv, pval_hbm.at[pl.ds(w0 + off, SUB)])
                pltpu.sync_copy(sp_n.at[pl.ds(off, SUB)], sn)
                pltpu.sync_copy(sn, pn_hbm.at[pl.ds(w0 + off, SUB)])
                pltpu.sync_copy(sp_dst.at[pl.ds(off, SUB)], sd)
                pltpu.sync_copy(sd, pdst_hbm.at[pl.ds(w0 + off, SUB)])
                return 0
            lax.fori_loop(0, HPT // SUB, dump, 0)
            plsc.subcore_barrier()


@functools.lru_cache(maxsize=None)
def _make_part():
    return pl.kernel(
        _part_body2,
        out_type=(jax.ShapeDtypeStruct((MPAD,), jnp.float32),
                  jax.ShapeDtypeStruct((MPAD,), jnp.float32),
                  jax.ShapeDtypeStruct((MPAD,), jnp.int32),
                  jax.ShapeDtypeStruct((48,), jnp.int32)),
        mesh=_mesh(),
        compiler_params=_scparams(),
        scratch_types=[
            pltpu.VMEM_SHARED((NT, 128), jnp.int32),
            pltpu.VMEM_SHARED((HALF + 8,), jnp.float32),
            pltpu.VMEM_SHARED((HALF + 8,), jnp.float32),
            pltpu.VMEM_SHARED((HALF + 8,), jnp.int32),
            pltpu.VMEM_SHARED((PAD,), jnp.float32),
            pltpu.VMEM_SHARED((PAD,), jnp.float32),
            pltpu.VMEM((WSUB,), jnp.int32),
            pltpu.VMEM((WSUB,), jnp.int32),
            pltpu.VMEM((WSUB,), jnp.int32),
            pltpu.VMEM((WSUB,), jnp.float32),
            pltpu.VMEM((WSUB,), jnp.float32),
            pltpu.VMEM((WSUB,), jnp.float32),
            pltpu.VMEM((PAD // 64,), jnp.int32),
            pltpu.VMEM((128,), jnp.int32),
            pltpu.VMEM((32,), jnp.int32),
            pltpu.VMEM((16,), jnp.int32),
            pltpu.VMEM((NT, 128), jnp.int32),
            pltpu.VMEM((48,), jnp.int32),
        ],
        name="sc_edge_partition",
    )


def _acc_body(pval_hbm, pn_hbm, pdst_hbm, rstart_hbm, w1_hbm, b1_hbm, h_hbm,
              dbuf, valbuf, nbuf, wbuf, bbuf, block, rsbuf):
    c = lax.axis_index("c")
    t = lax.axis_index("s")
    w = t * 2 + c
    d0 = w * RW

    pltpu.sync_copy(w1_hbm, wbuf)
    pltpu.sync_copy(b1_hbm, bbuf)
    pltpu.sync_copy(rstart_hbm, rsbuf)
    iota = lax.iota(jnp.int32, 16)
    rv = plsc.load_gather(rsbuf, [w + jnp.minimum(iota, 1)])
    start = rv[0]
    end = rv[1]
    s8 = start & ~7
    nwin = (end - s8 + SUB - 1) >> 11
    w1 = [wbuf[pl.ds(k * 16, 16)] for k in range(4)]
    b1v = [bbuf[pl.ds(k * 16, 16)] for k in range(4)]

    def zb(i, _):
        block[pl.ds(i * 16, 16)] = jnp.zeros((16,), jnp.float32)
        return 0
    lax.fori_loop(0, RW * F1 // 16, zb, 0)

    def win_step(jw, _):
        wlo = pl.multiple_of(s8 + jw * SUB, 8)
        pltpu.sync_copy(pdst_hbm.at[pl.ds(wlo, SUB)], dbuf)
        pltpu.sync_copy(pval_hbm.at[pl.ds(wlo, SUB)], valbuf)
        pltpu.sync_copy(pn_hbm.at[pl.ds(wlo, SUB)], nbuf)
        lo = jnp.maximum(start - wlo, 0)
        hi = jnp.minimum(end - wlo, SUB)
        vlo = lo >> 4
        vhi = (hi + 15) >> 4

        def vec_step(jv, _2):
            d16 = dbuf[pl.ds(jv * 16, 16)]
            v16 = valbuf[pl.ds(jv * 16, 16)]
            n16 = nbuf[pl.ds(jv * 16, 16)]
            gidx = wlo + jv * 16
            for k in range(16):
                pos = gidx + k
                valid = jnp.logical_and(pos >= start, pos < end)

                @pl.when(valid)
                def _(d=d16[k], val=v16[k], n=n16[k]):
                    off = (d - d0) * F1
                    for q in range(4):
                        r = block[pl.ds(off + q * 16, 16)]
                        block[pl.ds(off + q * 16, 16)] = r + (val * w1[q]) * n
            return 0
        lax.fori_loop(vlo, vhi, vec_step, 0)
        return 0
    lax.fori_loop(0, nwin, win_step, 0)

    def finwalk(nd, _):
        off = nd * F1
        for q in range(4):
            block[pl.ds(off + q * 16, 16)] = jnp.maximum(
                block[pl.ds(off + q * 16, 16)] + b1v[q], 0.0)
        return 0
    lax.fori_loop(0, RW, finwalk, 0)
    pltpu.sync_copy(block, h_hbm.at[pl.ds(d0 * F1, RW * F1)])


@functools.lru_cache(maxsize=None)
def _make_acc():
    return pl.kernel(
        _acc_body,
        out_type=jax.ShapeDtypeStruct((PAD * F1,), jnp.float32),
        mesh=_mesh(),
        compiler_params=_scparams(),
        scratch_types=[
            pltpu.VMEM((SUB,), jnp.int32),
            pltpu.VMEM((SUB,), jnp.float32),
            pltpu.VMEM((SUB,), jnp.float32),
            pltpu.VMEM((F1,), jnp.float32),
            pltpu.VMEM((F1,), jnp.float32),
            pltpu.VMEM((RW * F1,), jnp.float32),
            pltpu.VMEM((48,), jnp.int32),
        ],
        name="sc_conv1_accumulate",
    )


def _gsa_body(ntab, dst_hbm, src_hbm, tab_hbm, *rest):
    out_hbms = rest[:ntab]
    sp_accs, dbuf, sbuf, vbuf, tabs, zbuf = rest[ntab:]
    c = lax.axis_index("c")
    t = lax.axis_index("s")

    @pl.when(c == 0)
    def _core0():
        for q in range(ntab):
            pltpu.sync_copy(tab_hbm.at[q], tabs[q])

        def fill0(i, _):
            zbuf[pl.ds(i * 16, 16)] = jnp.zeros((16,), jnp.float32)
            return 0
        lax.fori_loop(0, CPT // 16, fill0, 0)
        for q in range(ntab):
            pltpu.sync_copy(zbuf, sp_accs[q].at[pl.ds(t * CPT, CPT)])
        plsc.subcore_barrier()

        def step(i, _):
            pltpu.sync_copy(dst_hbm.at[pl.ds(t * EPT + i * SUB, SUB)], dbuf)
            pltpu.sync_copy(src_hbm.at[pl.ds(t * EPT + i * SUB, SUB)], sbuf)
            for q in range(ntab):
                def gw(j, _2, _q=q):
                    s16 = sbuf[pl.ds(j * 16, 16)]
                    vbuf[pl.ds(j * 16, 16)] = plsc.load_gather(tabs[_q], [s16])
                    return 0
                lax.fori_loop(0, SUB // 16, gw, 0)
                pltpu.sync_copy(vbuf, sp_accs[q].at[dbuf], add=True)
            return 0
        lax.fori_loop(0, NSUB, step, 0)
        plsc.subcore_barrier()
        for q in range(ntab):
            pltpu.sync_copy(sp_accs[q].at[pl.ds(t * CPT, CPT)], zbuf)
            pltpu.sync_copy(zbuf, out_hbms[q].at[pl.ds(t * CPT, CPT)])


@functools.lru_cache(maxsize=None)
def _make_gsa(ntab):
    out_t = tuple(jax.ShapeDtypeStruct((PAD,), jnp.float32) for _ in range(ntab))
    return pl.kernel(
        functools.partial(_gsa_body, ntab),
        out_type=out_t[0] if ntab == 1 else out_t,
        mesh=_mesh(),
        compiler_params=_scparams(),
        scratch_types=[
            [pltpu.VMEM_SHARED((PAD,), jnp.float32) for _ in range(ntab)],
            pltpu.VMEM((SUB,), jnp.int32),
            pltpu.VMEM((SUB,), jnp.int32),
            pltpu.VMEM((SUB,), jnp.float32),
            [pltpu.VMEM((PAD,), jnp.float32) for _ in range(ntab)],
            pltpu.VMEM((CPT,), jnp.float32),
        ],
        name=f"sc_gather_scatter_add_{ntab}",
    )


NB = 1024
F2 = 128


def _dense_body(a_ref, b_ref, u_ref, v_ref, p2_ref, b2_ref, s2_ref):
    a = a_ref[...]
    b = b_ref[...]
    h2 = jnp.maximum(a[:, None] * u_ref[...][None, :] +
                     b[:, None] * v_ref[...][None, :] + b2_ref[...][None, :], 0.0)
    t2 = jnp.sum(h2 * p2_ref[...][None, :], axis=1)
    s2_ref[...] = jnp.tanh(t2)


def _dense_call(A, B, u, v, p2s, b2):
    return pl.pallas_call(
        _dense_body,
        grid=(PAD // NB,),
        in_specs=[
            pl.BlockSpec((NB,), lambda i: (i,)),
            pl.BlockSpec((NB,), lambda i: (i,)),
            pl.BlockSpec((F2,), lambda i: (0,)),
            pl.BlockSpec((F2,), lambda i: (0,)),
            pl.BlockSpec((F2,), lambda i: (0,)),
            pl.BlockSpec((F2,), lambda i: (0,)),
        ],
        out_specs=pl.BlockSpec((NB,), lambda i: (i,)),
        out_shape=jax.ShapeDtypeStruct((PAD,), jnp.float32),
    )(A, B, u, v, p2s, b2)


def _pool_body(a_ref, b_ref, u_ref, v_ref, b2_ref, s2_ref, m2_ref, g_ref):
    i = pl.program_id(0)

    @pl.when(i == 0)
    def _():
        g_ref[...] = jnp.full((1, F2), jnp.finfo(jnp.float32).min, jnp.float32)
    a = a_ref[...]
    b = b_ref[...]
    h2 = jnp.maximum(a[:, None] * u_ref[...][None, :] +
                     b[:, None] * v_ref[...][None, :] + b2_ref[...][None, :], 0.0)
    h2g = h2 * s2_ref[...][:, None]
    neg = jnp.finfo(jnp.float32).min
    vals = jnp.where(m2_ref[...][:, None] > 0, h2g, neg)
    g_ref[...] = jnp.maximum(g_ref[...], jnp.max(vals, axis=0, keepdims=True))


def _pool_call(A, B, u, v, b2, s2, m2):
    return pl.pallas_call(
        _pool_body,
        grid=(PAD // NB,),
        in_specs=[
            pl.BlockSpec((NB,), lambda i: (i,)),
            pl.BlockSpec((NB,), lambda i: (i,)),
            pl.BlockSpec((F2,), lambda i: (0,)),
            pl.BlockSpec((F2,), lambda i: (0,)),
            pl.BlockSpec((F2,), lambda i: (0,)),
            pl.BlockSpec((NB,), lambda i: (i,)),
            pl.BlockSpec((NB,), lambda i: (i,)),
        ],
        out_specs=pl.BlockSpec((1, F2), lambda i: (0, 0)),
        out_shape=jax.ShapeDtypeStruct((1, F2), jnp.float32),
    )(A, B, u, v, b2, s2, m2)


def kernel(x, edge_index, batch, W1, b1, p1, W2, b2, p2, fcW1, fcb1, fcW2, fcb2):
    N = x.shape[0]
    src0, dst0 = edge_index[0], edge_index[1]
    loop = jnp.arange(N, dtype=src0.dtype)
    padw = jnp.full((MPAD - M_EDGES,), PAD - 1, src0.dtype)
    srcp = jnp.concatenate([src0, loop, padw]).astype(jnp.int32)
    dstp = jnp.concatenate([dst0, loop, padw]).astype(jnp.int32)

    degp = _make_deg()(dstp)
    deg = degp[:N]
    safe = jnp.where(deg > 0, deg, 1.0)
    dis = jnp.where(deg > 0, 1.0 / jnp.sqrt(safe), 0.0)
    xpad = jnp.concatenate([x[:, 0], jnp.zeros((PAD - N,), jnp.float32)])
    dispad = jnp.concatenate([dis, jnp.zeros((PAD - N,), jnp.float32)])
    btab = (jnp.arange(PAD // 64, dtype=jnp.int32) * 64) // RW
    pval, pn, pdst, rstart = _make_part()(dstp, srcp, xpad, dispad, btab)
    W1row = W1[0]
    hflat = _make_acc()(pval, pn, pdst, rstart, W1row, b1)
    h = hflat.reshape(PAD, F1)[:N]

    score1 = jnp.tanh((h @ p1) / jnp.sqrt(jnp.sum(p1 * p1)))
    k1 = int(math.ceil(0.8 * N))
    perm, mask1p = _sc_topk(score1, k1)
    mask1 = mask1p[:N]

    jp = jnp.argmax(W1row)
    jn = jnp.argmin(W1row)
    wp = W1row[jp]
    wn = W1row[jn]
    maxpos = jnp.where(wp > 0, jnp.take(h, jp, axis=1) / wp, 0.0)
    minneg = jnp.where(wn < 0, jnp.take(h, jn, axis=1) / wn, 0.0)
    gam = score1 * mask1
    a_nod = gam * maxpos
    b_nod = gam * minneg

    cnt = _make_gsa(1)(dstp, srcp, mask1p[None, :])[:N]
    deg2 = mask1 * (cnt + 1.0)
    safe2 = jnp.where(deg2 > 0, deg2, 1.0)
    dis2 = jnp.where(deg2 > 0, 1.0 / jnp.sqrt(safe2), 0.0)
    at_ = a_nod * dis2
    bt_ = b_nod * dis2
    zpadN = jnp.zeros((PAD - N,), jnp.float32)
    tabs = jnp.stack([jnp.concatenate([at_, zpadN]), jnp.concatenate([bt_, zpadN])])
    sumA, sumB = _make_gsa(2)(dstp, srcp, tabs)
    A = dis2 * mask1 * (sumA[:N] + at_)
    B = dis2 * mask1 * (sumB[:N] + bt_)

    W1p = jnp.maximum(W1row, 0.0)
    W1m = jnp.minimum(W1row, 0.0)
    u = W1p @ W2
    v = W1m @ W2
    p2s = p2 / jnp.sqrt(jnp.sum(p2 * p2))
    Ap = jnp.concatenate([A, zpadN])
    Bp = jnp.concatenate([B, zpadN])
    score2p = _dense_call(Ap, Bp, u, v, p2s, b2)
    score2 = score2p[:N]

    k2 = int(math.ceil(0.8 * k1))
    neg = jnp.finfo(x.dtype).min
    _perm2, mask2p = _sc_topk(jnp.where(mask1 > 0, score2, neg), k2)

    g = _pool_call(Ap, Bp, u, v, b2, score2p, mask2p)

    z = jax.nn.relu(g @ fcW1 + fcb1)
    z = z @ fcW2 + fcb2
    return jax.nn.log_softmax(z, axis=1), perm


def _conv_test(dstp, srcp, xpad, dispad, btab, w1, b1):
    degp = _make_deg()(dstp)
    pval, pn, pdst, rstart = _make_part()(dstp, srcp, xpad, dispad, btab)
    hflat = _make_acc()(pval, pn, pdst, rstart, w1, b1)
    cnt = _make_gsa(1)(dstp, srcp, xpad[None, :])
    sums = _make_gsa(2)(dstp, srcp, jnp.stack([xpad, dispad]))
    u = jnp.zeros((F2,), jnp.float32)
    s2 = _dense_call(xpad, dispad, u, u, u, u)
    g = _pool_call(xpad, dispad, u, u, u, s2, s2)
    return degp, hflat, cnt, sums, s2, g


def _conv_test_args():
    import numpy as np
    return (np.zeros((MPAD,), np.int32), np.zeros((MPAD,), np.int32),
            np.zeros((PAD,), np.float32), np.zeros((PAD,), np.float32),
            np.zeros((PAD // 64,), np.int32), np.zeros((F1,), np.float32),
            np.zeros((F1,), np.float32))


# speedup vs baseline: 1.0485x; 1.0485x over previous
import functools
import math

import jax
import jax.numpy as jnp
from jax import lax
from jax.experimental import pallas as pl
from jax.experimental.pallas import tpu as pltpu
from jax.experimental.pallas import tpu_sc as plsc

N_NODES = 50000
PAD = 50176
NT = 16
CPT = PAD // NT
NVEC = CPT // 16
NR = 32
RW = PAD // NR
E_REAL = 800000
M_EDGES = E_REAL + N_NODES
MPAD = 851968
EPT = MPAD // NT
SUB = 2048
HALF = MPAD // 2
HPT = HALF // NT
NSUB = EPT // SUB
F1 = 64


def _take16(v, idx):
    return v.at[idx].get(mode="promise_in_bounds")


def _mesh():
    return plsc.VectorSubcoreMesh(core_axis_name="c", subcore_axis_name="s")


def _scparams():
    return pltpu.CompilerParams(needs_layout_passes=False)


def _sort_body(k_keep, keys_hbm, idx_hbm, mask_hbm,
               spA_k, spA_v, spB_k, spB_v, sp_hist, sp_mask,
               kbuf, vbuf, posbuf, mibuf, onesbuf, zbuf, hbuf, hist, base, tmp16):
    c = lax.axis_index("c")
    t = lax.axis_index("s")

    @pl.when(c == 0)
    def _core0():
        iota = lax.iota(jnp.int32, 16)

        def fill_const(buf, val16):
            def step(i, _):
                buf[pl.ds(i * 16, 16)] = val16
                return 0
            lax.fori_loop(0, NVEC, step, 0)

        fill_const(onesbuf, jnp.full((16,), 1.0, jnp.float32))
        fill_const(zbuf, jnp.zeros((16,), jnp.float32))
        pltpu.sync_copy(zbuf, sp_mask.at[pl.ds(t * CPT, CPT)])
        pltpu.sync_copy(keys_hbm.at[pl.ds(t * CPT, CPT)], kbuf)

        def fill_v(i, _):
            vbuf[pl.ds(i * 16, 16)] = t * CPT + i * 16 + iota
            return 0
        lax.fori_loop(0, NVEC, fill_v, 0)
        plsc.subcore_barrier()

        def vec_ranks(kk, sh):
            d16 = ((kk >> sh) & 0xFF).astype(jnp.int32)
            skey = (d16 << 4) | iota
            sk, _ = plsc.sort_key_val(skey, iota)
            ds = sk >> 4
            lanes = sk & 15
            prev = jnp.where(iota == 0, 256, _take16(ds, jnp.maximum(iota - 1, 0)))
            nxt = jnp.where(iota == 15, 256, _take16(ds, jnp.minimum(iota + 1, 15)))
            isfirst = ds != prev
            islast = ds != nxt
            starts = plsc.cummax(jnp.where(isfirst, iota, 0))
            rank = iota - starts
            return ds, lanes, rank, islast

        for p in range(4):
            sh = 8 * p
            src_k, src_v = (spA_k, spA_v) if p % 2 == 0 else (spB_k, spB_v)
            dst_k, dst_v = (spB_k, spB_v) if p % 2 == 0 else (spA_k, spA_v)
            if p > 0:
                pltpu.sync_copy(src_k.at[pl.ds(t * CPT, CPT)], kbuf)
                pltpu.sync_copy(src_v.at[pl.ds(t * CPT, CPT)], vbuf)

            def zh(i, _):
                hist[pl.ds(i * 16, 16)] = jnp.zeros((16,), jnp.int32)
                return 0
            lax.fori_loop(0, 16, zh, 0)

            def hwalk(i, _):
                kk = kbuf[pl.ds(i * 16, 16)]
                ds, _lanes, rank, islast = vec_ranks(kk, sh)
                old = plsc.load_gather(hist, [ds])
                plsc.store_scatter(hist, [ds], old + rank + 1, mask=islast)
                return 0
            lax.fori_loop(0, NVEC, hwalk, 0)
            pltpu.sync_copy(hist, sp_hist.at[t])
            plsc.subcore_barrier()
            pltpu.sync_copy(sp_hist, hbuf)

            def scanstep(dv, carry):
                def acc_t(tt, a):
                    return a + hbuf[tt, pl.ds(dv * 16, 16)]
                totv = lax.fori_loop(0, 16, acc_t, jnp.zeros((16,), jnp.int32))
                cs = plsc.cumsum(totv)
                excl = cs - totv + carry
                pfxv = lax.fori_loop(0, t, acc_t, jnp.zeros((16,), jnp.int32))
                base[pl.ds(dv * 16, 16)] = excl + pfxv
                return carry + cs[15]
            lax.fori_loop(0, 16, scanstep, jnp.int32(0))

            def swalk(i, _):
                kk = kbuf[pl.ds(i * 16, 16)]
                ds, lanes, rank, islast = vec_ranks(kk, sh)
                bs = plsc.load_gather(base, [ds])
                pos_s = bs + rank
                plsc.store_scatter(base, [ds], pos_s + 1, mask=islast)
                plsc.store_scatter(tmp16, [lanes], pos_s)
                posbuf[pl.ds(i * 16, 16)] = tmp16[pl.ds(0, 16)]
                return 0
            lax.fori_loop(0, NVEC, swalk, 0)
            pltpu.sync_copy(kbuf, dst_k.at[posbuf])
            pltpu.sync_copy(vbuf, dst_v.at[posbuf])
            plsc.subcore_barrier()

        pltpu.sync_copy(spA_v.at[pl.ds(t * CPT, CPT)], vbuf)
        pltpu.sync_copy(vbuf, idx_hbm.at[pl.ds(t * CPT, CPT)])

        def mwalk(i, _):
            v16 = vbuf[pl.ds(i * 16, 16)]
            gpos = t * CPT + i * 16 + iota
            mibuf[pl.ds(i * 16, 16)] = jnp.where(gpos < k_keep, v16, N_NODES)
            return 0
        lax.fori_loop(0, NVEC, mwalk, 0)
        pltpu.sync_copy(onesbuf, sp_mask.at[mibuf])
        plsc.subcore_barrier()
        pltpu.sync_copy(sp_mask.at[pl.ds(t * CPT, CPT)], zbuf)
        pltpu.sync_copy(zbuf, mask_hbm.at[pl.ds(t * CPT, CPT)])


@functools.lru_cache(maxsize=None)
def _make_sort(k_keep):
    return pl.kernel(
        functools.partial(_sort_body, k_keep),
        out_type=(jax.ShapeDtypeStruct((PAD,), jnp.int32),
                  jax.ShapeDtypeStruct((PAD,), jnp.float32)),
        mesh=_mesh(),
        compiler_params=_scparams(),
        scratch_types=[
            pltpu.VMEM_SHARED((PAD,), jnp.uint32),
            pltpu.VMEM_SHARED((PAD,), jnp.int32),
            pltpu.VMEM_SHARED((PAD,), jnp.uint32),
            pltpu.VMEM_SHARED((PAD,), jnp.int32),
            pltpu.VMEM_SHARED((NT, 256), jnp.int32),
            pltpu.VMEM_SHARED((PAD,), jnp.float32),
            pltpu.VMEM((CPT,), jnp.uint32),
            pltpu.VMEM((CPT,), jnp.int32),
            pltpu.VMEM((CPT,), jnp.int32),
            pltpu.VMEM((CPT,), jnp.int32),
            pltpu.VMEM((CPT,), jnp.float32),
            pltpu.VMEM((CPT,), jnp.float32),
            pltpu.VMEM((NT, 256), jnp.int32),
            pltpu.VMEM((256,), jnp.int32),
            pltpu.VMEM((256,), jnp.int32),
            pltpu.VMEM((16,), jnp.int32),
        ],
        name=f"sc_topk_sort_{k_keep}",
    )


def _desc_keys(score):
    b = lax.bitcast_convert_type(score, jnp.uint32)
    asc = jnp.where(b >> 31 != 0, ~b, b | jnp.uint32(0x80000000))
    desc = ~asc
    return jnp.concatenate([desc, jnp.full((PAD - N_NODES,), 0xFFFFFFFF, jnp.uint32)])


def _sc_topk(score, k_keep):
    idx_pad, mask_pad = _make_sort(k_keep)(_desc_keys(score))
    return idx_pad[:k_keep], mask_pad


def _deg_body2(dst_hbm, deg_hbm, sp_deg, dbuf, vbuf, zbuf):
    c = lax.axis_index("c")
    t = lax.axis_index("s")

    @pl.when(c == 0)
    def _core0():
        def fill1(i, _):
            vbuf[pl.ds(i * 16, 16)] = jnp.full((16,), 1.0, jnp.float32)
            return 0
        lax.fori_loop(0, SUB // 16, fill1, 0)

        def fill0(i, _):
            zbuf[pl.ds(i * 16, 16)] = jnp.zeros((16,), jnp.float32)
            return 0
        lax.fori_loop(0, CPT // 16, fill0, 0)
        pltpu.sync_copy(zbuf, sp_deg.at[pl.ds(t * CPT, CPT)])
        plsc.subcore_barrier()

        def step(i, _):
            pltpu.sync_copy(dst_hbm.at[pl.ds(t * EPT + i * SUB, SUB)], dbuf)
            pltpu.sync_copy(vbuf, sp_deg.at[dbuf], add=True)
            return 0
        lax.fori_loop(0, NSUB, step, 0)
        plsc.subcore_barrier()
        pltpu.sync_copy(sp_deg.at[pl.ds(t * CPT, CPT)], zbuf)
        pltpu.sync_copy(zbuf, deg_hbm.at[pl.ds(t * CPT, CPT)])


@functools.lru_cache(maxsize=None)
def _make_deg():
    return pl.kernel(
        _deg_body2,
        out_type=jax.ShapeDtypeStruct((PAD,), jnp.float32),
        mesh=_mesh(),
        compiler_params=_scparams(),
        scratch_types=[
            pltpu.VMEM_SHARED((PAD,), jnp.float32),
            pltpu.VMEM((SUB,), jnp.int32),
            pltpu.VMEM((SUB,), jnp.float32),
            pltpu.VMEM((CPT,), jnp.float32),
        ],
        name="sc_deg_hist",
    )


def _bucket_of(d16, btab, iota):
    q = d16 >> 6
    tb = plsc.load_gather(btab, [q])
    return tb + jnp.where(d16 >= (tb + 1) * RW, 1, 0)


def _part_body2(dst_hbm, src_hbm, x_hbm, dis_hbm, btab_hbm,
                pval_hbm, pn_hbm, pdst_hbm, rstart_hbm,
                sp_hist, sp_val, sp_n, sp_dst, sp_x, sp_dis,
                dbuf, sbuf, posbuf, valbuf, nbuf, dsdbuf,
                btab, hist, base, tmp16, hbuf, rsbuf):
    c = lax.axis_index("c")
    t = lax.axis_index("s")

    @pl.when(c == 0)
    def _core0():
        iota = lax.iota(jnp.int32, 16)
        pltpu.sync_copy(btab_hbm, btab)

        @pl.when(t == 0)
        def _load_tabs():
            pltpu.sync_copy(x_hbm, sp_x)
            pltpu.sync_copy(dis_hbm, sp_dis)

        def zh(i, _):
            hist[pl.ds(i * 16, 16)] = jnp.zeros((16,), jnp.int32)
            return 0
        lax.fori_loop(0, 8, zh, 0)

        def vec_ranks32(b16):
            skey = (b16 << 4) | iota
            sk, _ = plsc.sort_key_val(skey, iota)
            ds = sk >> 4
            lanes = sk & 15
            prev = jnp.where(iota == 0, 64, _take16(ds, jnp.maximum(iota - 1, 0)))
            nxt = jnp.where(iota == 15, 64, _take16(ds, jnp.minimum(iota + 1, 15)))
            isfirst = ds != prev
            islast = ds != nxt
            starts = plsc.cummax(jnp.where(isfirst, iota, 0))
            rank = iota - starts
            return ds, lanes, rank, islast

        def hsub(i, _):
            pltpu.sync_copy(dst_hbm.at[pl.ds(t * EPT + i * SUB, SUB)], dbuf)

            def hwalk(j, _2):
                d16 = dbuf[pl.ds(j * 16, 16)]
                b16 = _bucket_of(d16, btab, iota)
                ds, _l, rank, islast = vec_ranks32(b16)
                old = plsc.load_gather(hist, [ds])
                plsc.store_scatter(hist, [ds], old + rank + 1, mask=islast)
                return 0
            lax.fori_loop(0, SUB // 16, hwalk, 0)
            return 0
        lax.fori_loop(0, NSUB, hsub, 0)
        pltpu.sync_copy(hist, sp_hist.at[t])
        plsc.subcore_barrier()
        pltpu.sync_copy(sp_hist, hbuf)

        def scanstep(bv, carry):
            def acc_t(tt, a):
                return a + hbuf[tt, pl.ds(bv * 16, 16)]
            totv = lax.fori_loop(0, NT, acc_t, jnp.zeros((16,), jnp.int32))
            cs = plsc.cumsum(totv)
            excl = cs - totv + carry
            pfxv = lax.fori_loop(0, t, acc_t, jnp.zeros((16,), jnp.int32))
            base[pl.ds(bv * 16, 16)] = excl + pfxv
            rsbuf[pl.ds(bv * 16, 16)] = excl
            return carry + cs[15]
        total = lax.fori_loop(0, 2, scanstep, jnp.int32(0))

        @pl.when(t == 0)
        def _wr_rstart():
            rsbuf[pl.ds(32, 16)] = jnp.full((16,), total, jnp.int32)
            pltpu.sync_copy(rsbuf, rstart_hbm)

        for wpass in range(2):
            w0 = wpass * HALF

            def rebase(bv, carry):
                def acc_t(tt, a):
                    return a + hbuf[tt, pl.ds(bv * 16, 16)]
                totv = lax.fori_loop(0, NT, acc_t, jnp.zeros((16,), jnp.int32))
                cs = plsc.cumsum(totv)
                excl = cs - totv + carry
                pfxv = lax.fori_loop(0, t, acc_t, jnp.zeros((16,), jnp.int32))
                base[pl.ds(bv * 16, 16)] = excl + pfxv
                return carry + cs[15]
            lax.fori_loop(0, 2, rebase, jnp.int32(0))

            def psub(i, _):
                pltpu.sync_copy(dst_hbm.at[pl.ds(t * EPT + i * SUB, SUB)], dbuf)
                pltpu.sync_copy(src_hbm.at[pl.ds(t * EPT + i * SUB, SUB)], sbuf)
                pltpu.sync_copy(sp_x.at[sbuf], valbuf)
                pltpu.sync_copy(sp_dis.at[sbuf], nbuf)
                pltpu.sync_copy(sp_dis.at[dbuf], dsdbuf)

                def pwalk(j, _2):
                    d16 = dbuf[pl.ds(j * 16, 16)]
                    b16 = _bucket_of(d16, btab, iota)
                    ds, lanes, rank, islast = vec_ranks32(b16)
                    bs = plsc.load_gather(base, [ds])
                    pos_s = bs + rank
                    plsc.store_scatter(base, [ds], pos_s + 1, mask=islast)
                    plsc.store_scatter(tmp16, [lanes], pos_s)
                    pos16 = tmp16[pl.ds(0, 16)]
                    loc = pos16 - w0
                    inwin = jnp.logical_and(pos16 >= w0, loc < HALF)
                    posbuf[pl.ds(j * 16, 16)] = jnp.where(inwin, loc, HALF)
                    nbuf[pl.ds(j * 16, 16)] = nbuf[pl.ds(j * 16, 16)] * dsdbuf[pl.ds(j * 16, 16)]
                    return 0
                lax.fori_loop(0, SUB // 16, pwalk, 0)
                pltpu.sync_copy(valbuf, sp_val.at[posbuf])
                pltpu.sync_copy(nbuf, sp_n.at[posbuf])
                pltpu.sync_copy(dbuf, sp_dst.at[posbuf])
                return 0
            lax.fori_loop(0, NSUB, psub, 0)
            plsc.subcore_barrier()

            def dump(i, _):
                off = t * HPT + i * SUB
                pltpu.sync_copy(sp_val.at[pl.ds(off, SUB)], valbuf)
                pltpu.sync_copy(valbuf, pval_hbm.at[pl.ds(w0 + off, SUB)])
                pltpu.sync_copy(sp_n.at[pl.ds(off, SUB)], nbuf)
                pltpu.sync_copy(nbuf, pn_hbm.at[pl.ds(w0 + off, SUB)])
                pltpu.sync_copy(sp_dst.at[pl.ds(off, SUB)], dbuf)
                pltpu.sync_copy(dbuf, pdst_hbm.at[pl.ds(w0 + off, SUB)])
                return 0
            lax.fori_loop(0, HPT // SUB, dump, 0)
            plsc.subcore_barrier()


@functools.lru_cache(maxsize=None)
def _make_part():
    return pl.kernel(
        _part_body2,
        out_type=(jax.ShapeDtypeStruct((MPAD,), jnp.float32),
                  jax.ShapeDtypeStruct((MPAD,), jnp.float32),
                  jax.ShapeDtypeStruct((MPAD,), jnp.int32),
                  jax.ShapeDtypeStruct((48,), jnp.int32)),
        mesh=_mesh(),
        compiler_params=_scparams(),
        scratch_types=[
            pltpu.VMEM_SHARED((NT, 128), jnp.int32),
            pltpu.VMEM_SHARED((HALF + 8,), jnp.float32),
            pltpu.VMEM_SHARED((HALF + 8,), jnp.float32),
            pltpu.VMEM_SHARED((HALF + 8,), jnp.int32),
            pltpu.VMEM_SHARED((PAD,), jnp.float32),
            pltpu.VMEM_SHARED((PAD,), jnp.float32),
            pltpu.VMEM((SUB,), jnp.int32),
            pltpu.VMEM((SUB,), jnp.int32),
            pltpu.VMEM((SUB,), jnp.int32),
            pltpu.VMEM((SUB,), jnp.float32),
            pltpu.VMEM((SUB,), jnp.float32),
            pltpu.VMEM((SUB,), jnp.float32),
            pltpu.VMEM((PAD // 64,), jnp.int32),
            pltpu.VMEM((128,), jnp.int32),
            pltpu.VMEM((32,), jnp.int32),
            pltpu.VMEM((16,), jnp.int32),
            pltpu.VMEM((NT, 128), jnp.int32),
            pltpu.VMEM((48,), jnp.int32),
        ],
        name="sc_edge_partition",
    )


def _acc_body(pval_hbm, pn_hbm, pdst_hbm, rstart_hbm, w1_hbm, b1_hbm, h_hbm,
              dbuf, valbuf, nbuf, wbuf, bbuf, block, rsbuf):
    c = lax.axis_index("c")
    t = lax.axis_index("s")
    w = t * 2 + c
    d0 = w * RW

    pltpu.sync_copy(w1_hbm, wbuf)
    pltpu.sync_copy(b1_hbm, bbuf)
    pltpu.sync_copy(rstart_hbm, rsbuf)
    iota = lax.iota(jnp.int32, 16)
    rv = plsc.load_gather(rsbuf, [w + jnp.minimum(iota, 1)])
    start = rv[0]
    end = rv[1]
    s8 = start & ~7
    nwin = (end - s8 + SUB - 1) >> 11
    w1 = [wbuf[pl.ds(k * 16, 16)] for k in range(4)]
    b1v = [bbuf[pl.ds(k * 16, 16)] for k in range(4)]

    def zb(i, _):
        block[pl.ds(i * 16, 16)] = jnp.zeros((16,), jnp.float32)
        return 0
    lax.fori_loop(0, RW * F1 // 16, zb, 0)

    def win_step(jw, _):
        wlo = pl.multiple_of(s8 + jw * SUB, 8)
        pltpu.sync_copy(pdst_hbm.at[pl.ds(wlo, SUB)], dbuf)
        pltpu.sync_copy(pval_hbm.at[pl.ds(wlo, SUB)], valbuf)
        pltpu.sync_copy(pn_hbm.at[pl.ds(wlo, SUB)], nbuf)
        lo = jnp.maximum(start - wlo, 0)
        hi = jnp.minimum(end - wlo, SUB)
        vlo = lo >> 4
        vhi = (hi + 15) >> 4

        def vec_step(jv, _2):
            d16 = dbuf[pl.ds(jv * 16, 16)]
            v16 = valbuf[pl.ds(jv * 16, 16)]
            n16 = nbuf[pl.ds(jv * 16, 16)]
            gidx = wlo + jv * 16
            for k in range(16):
                pos = gidx + k
                valid = jnp.logical_and(pos >= start, pos < end)

                @pl.when(valid)
                def _(d=d16[k], val=v16[k], n=n16[k]):
                    off = (d - d0) * F1
                    for q in range(4):
                        r = block[pl.ds(off + q * 16, 16)]
                        block[pl.ds(off + q * 16, 16)] = r + (val * w1[q]) * n
            return 0
        lax.fori_loop(vlo, vhi, vec_step, 0)
        return 0
    lax.fori_loop(0, nwin, win_step, 0)

    def finwalk(nd, _):
        off = nd * F1
        for q in range(4):
            block[pl.ds(off + q * 16, 16)] = jnp.maximum(
                block[pl.ds(off + q * 16, 16)] + b1v[q], 0.0)
        return 0
    lax.fori_loop(0, RW, finwalk, 0)
    pltpu.sync_copy(block, h_hbm.at[pl.ds(d0 * F1, RW * F1)])


@functools.lru_cache(maxsize=None)
def _make_acc():
    return pl.kernel(
        _acc_body,
        out_type=jax.ShapeDtypeStruct((PAD * F1,), jnp.float32),
        mesh=_mesh(),
        compiler_params=_scparams(),
        scratch_types=[
            pltpu.VMEM((SUB,), jnp.int32),
            pltpu.VMEM((SUB,), jnp.float32),
            pltpu.VMEM((SUB,), jnp.float32),
            pltpu.VMEM((F1,), jnp.float32),
            pltpu.VMEM((F1,), jnp.float32),
            pltpu.VMEM((RW * F1,), jnp.float32),
            pltpu.VMEM((48,), jnp.int32),
        ],
        name="sc_conv1_accumulate",
    )


def _gsa_body(ntab, dst_hbm, src_hbm, tab_hbm, *rest):
    out_hbms = rest[:ntab]
    sp_accs, dbuf, sbuf, vbuf, tabs, zbuf = rest[ntab:]
    c = lax.axis_index("c")
    t = lax.axis_index("s")

    @pl.when(c == 0)
    def _core0():
        for q in range(ntab):
            pltpu.sync_copy(tab_hbm.at[q], tabs[q])

        def fill0(i, _):
            zbuf[pl.ds(i * 16, 16)] = jnp.zeros((16,), jnp.float32)
            return 0
        lax.fori_loop(0, CPT // 16, fill0, 0)
        for q in range(ntab):
            pltpu.sync_copy(zbuf, sp_accs[q].at[pl.ds(t * CPT, CPT)])
        plsc.subcore_barrier()

        def step(i, _):
            pltpu.sync_copy(dst_hbm.at[pl.ds(t * EPT + i * SUB, SUB)], dbuf)
            pltpu.sync_copy(src_hbm.at[pl.ds(t * EPT + i * SUB, SUB)], sbuf)
            for q in range(ntab):
                def gw(j, _2, _q=q):
                    s16 = sbuf[pl.ds(j * 16, 16)]
                    vbuf[pl.ds(j * 16, 16)] = plsc.load_gather(tabs[_q], [s16])
                    return 0
                lax.fori_loop(0, SUB // 16, gw, 0)
                pltpu.sync_copy(vbuf, sp_accs[q].at[dbuf], add=True)
            return 0
        lax.fori_loop(0, NSUB, step, 0)
        plsc.subcore_barrier()
        for q in range(ntab):
            pltpu.sync_copy(sp_accs[q].at[pl.ds(t * CPT, CPT)], zbuf)
            pltpu.sync_copy(zbuf, out_hbms[q].at[pl.ds(t * CPT, CPT)])


@functools.lru_cache(maxsize=None)
def _make_gsa(ntab):
    out_t = tuple(jax.ShapeDtypeStruct((PAD,), jnp.float32) for _ in range(ntab))
    return pl.kernel(
        functools.partial(_gsa_body, ntab),
        out_type=out_t[0] if ntab == 1 else out_t,
        mesh=_mesh(),
        compiler_params=_scparams(),
        scratch_types=[
            [pltpu.VMEM_SHARED((PAD,), jnp.float32) for _ in range(ntab)],
            pltpu.VMEM((SUB,), jnp.int32),
            pltpu.VMEM((SUB,), jnp.int32),
            pltpu.VMEM((SUB,), jnp.float32),
            [pltpu.VMEM((PAD,), jnp.float32) for _ in range(ntab)],
            pltpu.VMEM((CPT,), jnp.float32),
        ],
        name=f"sc_gather_scatter_add_{ntab}",
    )


NB = 1024
F2 = 128


def _dense_body(a_ref, b_ref, u_ref, v_ref, p2_ref, b2_ref, s2_ref):
    a = a_ref[...]
    b = b_ref[...]
    h2 = jnp.maximum(a[:, None] * u_ref[...][None, :] +
                     b[:, None] * v_ref[...][None, :] + b2_ref[...][None, :], 0.0)
    t2 = jnp.sum(h2 * p2_ref[...][None, :], axis=1)
    s2_ref[...] = jnp.tanh(t2)


def _dense_call(A, B, u, v, p2s, b2):
    return pl.pallas_call(
        _dense_body,
        grid=(PAD // NB,),
        in_specs=[
            pl.BlockSpec((NB,), lambda i: (i,)),
            pl.BlockSpec((NB,), lambda i: (i,)),
            pl.BlockSpec((F2,), lambda i: (0,)),
            pl.BlockSpec((F2,), lambda i: (0,)),
            pl.BlockSpec((F2,), lambda i: (0,)),
            pl.BlockSpec((F2,), lambda i: (0,)),
        ],
        out_specs=pl.BlockSpec((NB,), lambda i: (i,)),
        out_shape=jax.ShapeDtypeStruct((PAD,), jnp.float32),
    )(A, B, u, v, p2s, b2)


def _pool_body(a_ref, b_ref, u_ref, v_ref, b2_ref, s2_ref, m2_ref, g_ref):
    i = pl.program_id(0)

    @pl.when(i == 0)
    def _():
        g_ref[...] = jnp.full((1, F2), jnp.finfo(jnp.float32).min, jnp.float32)
    a = a_ref[...]
    b = b_ref[...]
    h2 = jnp.maximum(a[:, None] * u_ref[...][None, :] +
                     b[:, None] * v_ref[...][None, :] + b2_ref[...][None, :], 0.0)
    h2g = h2 * s2_ref[...][:, None]
    neg = jnp.finfo(jnp.float32).min
    vals = jnp.where(m2_ref[...][:, None] > 0, h2g, neg)
    g_ref[...] = jnp.maximum(g_ref[...], jnp.max(vals, axis=0, keepdims=True))


def _pool_call(A, B, u, v, b2, s2, m2):
    return pl.pallas_call(
        _pool_body,
        grid=(PAD // NB,),
        in_specs=[
            pl.BlockSpec((NB,), lambda i: (i,)),
            pl.BlockSpec((NB,), lambda i: (i,)),
            pl.BlockSpec((F2,), lambda i: (0,)),
            pl.BlockSpec((F2,), lambda i: (0,)),
            pl.BlockSpec((F2,), lambda i: (0,)),
            pl.BlockSpec((NB,), lambda i: (i,)),
            pl.BlockSpec((NB,), lambda i: (i,)),
        ],
        out_specs=pl.BlockSpec((1, F2), lambda i: (0, 0)),
        out_shape=jax.ShapeDtypeStruct((1, F2), jnp.float32),
    )(A, B, u, v, b2, s2, m2)


def kernel(x, edge_index, batch, W1, b1, p1, W2, b2, p2, fcW1, fcb1, fcW2, fcb2):
    N = x.shape[0]
    src0, dst0 = edge_index[0], edge_index[1]
    loop = jnp.arange(N, dtype=src0.dtype)
    padw = jnp.full((MPAD - M_EDGES,), PAD - 1, src0.dtype)
    srcp = jnp.concatenate([src0, loop, padw]).astype(jnp.int32)
    dstp = jnp.concatenate([dst0, loop, padw]).astype(jnp.int32)

    degp = _make_deg()(dstp)
    deg = degp[:N]
    safe = jnp.where(deg > 0, deg, 1.0)
    dis = jnp.where(deg > 0, 1.0 / jnp.sqrt(safe), 0.0)
    xpad = jnp.concatenate([x[:, 0], jnp.zeros((PAD - N,), jnp.float32)])
    dispad = jnp.concatenate([dis, jnp.zeros((PAD - N,), jnp.float32)])
    btab = (jnp.arange(PAD // 64, dtype=jnp.int32) * 64) // RW
    pval, pn, pdst, rstart = _make_part()(dstp, srcp, xpad, dispad, btab)
    W1row = W1[0]
    hflat = _make_acc()(pval, pn, pdst, rstart, W1row, b1)
    h = hflat.reshape(PAD, F1)[:N]

    score1 = jnp.tanh((h @ p1) / jnp.sqrt(jnp.sum(p1 * p1)))
    k1 = int(math.ceil(0.8 * N))
    perm, mask1p = _sc_topk(score1, k1)
    mask1 = mask1p[:N]

    jp = jnp.argmax(W1row)
    jn = jnp.argmin(W1row)
    wp = W1row[jp]
    wn = W1row[jn]
    maxpos = jnp.where(wp > 0, jnp.take(h, jp, axis=1) / wp, 0.0)
    minneg = jnp.where(wn < 0, jnp.take(h, jn, axis=1) / wn, 0.0)
    gam = score1 * mask1
    a_nod = gam * maxpos
    b_nod = gam * minneg

    cnt = _make_gsa(1)(dstp, srcp, mask1p[None, :])[:N]
    deg2 = mask1 * (cnt + 1.0)
    safe2 = jnp.where(deg2 > 0, deg2, 1.0)
    dis2 = jnp.where(deg2 > 0, 1.0 / jnp.sqrt(safe2), 0.0)
    at_ = a_nod * dis2
    bt_ = b_nod * dis2
    zpadN = jnp.zeros((PAD - N,), jnp.float32)
    tabs = jnp.stack([jnp.concatenate([at_, zpadN]), jnp.concatenate([bt_, zpadN])])
    sumA, sumB = _make_gsa(2)(dstp, srcp, tabs)
    A = dis2 * mask1 * (sumA[:N] + at_)
    B = dis2 * mask1 * (sumB[:N] + bt_)

    W1p = jnp.maximum(W1row, 0.0)
    W1m = jnp.minimum(W1row, 0.0)
    u = W1p @ W2
    v = W1m @ W2
    p2s = p2 / jnp.sqrt(jnp.sum(p2 * p2))
    Ap = jnp.concatenate([A, zpadN])
    Bp = jnp.concatenate([B, zpadN])
    score2p = _dense_call(Ap, Bp, u, v, p2s, b2)
    score2 = score2p[:N]

    k2 = int(math.ceil(0.8 * k1))
    neg = jnp.finfo(x.dtype).min
    _perm2, mask2p = _sc_topk(jnp.where(mask1 > 0, score2, neg), k2)

    g = _pool_call(Ap, Bp, u, v, b2, score2p, mask2p)

    z = jax.nn.relu(g @ fcW1 + fcb1)
    z = z @ fcW2 + fcb2
    return jax.nn.log_softmax(z, axis=1), perm


def _conv_test(dstp, srcp, xpad, dispad, btab, w1, b1):
    degp = _make_deg()(dstp)
    pval, pn, pdst, rstart = _make_part()(dstp, srcp, xpad, dispad, btab)
    hflat = _make_acc()(pval, pn, pdst, rstart, w1, b1)
    cnt = _make_gsa(1)(dstp, srcp, xpad[None, :])
    sums = _make_gsa(2)(dstp, srcp, jnp.stack([xpad, dispad]))
    u = jnp.zeros((F2,), jnp.float32)
    s2 = _dense_call(xpad, dispad, u, u, u, u)
    g = _pool_call(xpad, dispad, u, u, u, s2, s2)
    return degp, hflat, cnt, sums, s2, g


def _conv_test_args():
    import numpy as np
    return (np.zeros((MPAD,), np.int32), np.zeros((MPAD,), np.int32),
            np.zeros((PAD,), np.float32), np.zeros((PAD,), np.float32),
            np.zeros((PAD // 64,), np.int32), np.zeros((F1,), np.float32),
            np.zeros((F1,), np.float32))
